# baseline (speedup 1.0000x reference)
import functools

import jax
import jax.numpy as jnp
from jax import lax
from jax.experimental import pallas as pl
from jax.experimental.pallas import tpu as pltpu
from jax.experimental.pallas import tpu_sc as plsc

NUM_CORES = 2
NUM_SUBCORES = 16
NUM_WORKERS = NUM_CORES * NUM_SUBCORES


@jax.jit
def kernel(hyperparameters, table):
    num_emb, em_size = table.shape
    batch = hyperparameters.shape[0]
    idx = jnp.squeeze(hyperparameters, axis=1).astype(jnp.int32)
    b_per_w = batch // NUM_WORKERS

    mesh = plsc.VectorSubcoreMesh(core_axis_name="c", subcore_axis_name="s")

    @functools.partial(
        pl.kernel,
        mesh=mesh,
        out_type=jax.ShapeDtypeStruct((batch, em_size), jnp.float32),
        scratch_types=[
            pltpu.VMEM((b_per_w,), jnp.int32),
            pltpu.SemaphoreType.DMA,
        ],
    )
    def emb_lookup(table_hbm, idx_hbm, out_hbm, idx_v, sem):
        wid = lax.axis_index("s") * NUM_CORES + lax.axis_index("c")
        base = wid * b_per_w
        pltpu.sync_copy(idx_hbm.at[pl.ds(base, b_per_w)], idx_v)

        @pl.loop(0, b_per_w, step=16)
        def _(k):
            v = idx_v[pl.ds(k, 16)]
            for j in range(16):
                pltpu.async_copy(
                    table_hbm.at[v[j]], out_hbm.at[base + k + j], sem
                )

        pltpu.make_async_copy(
            table_hbm.at[pl.ds(0, b_per_w)],
            out_hbm.at[pl.ds(base, b_per_w)],
            sem,
        ).wait()

    return emb_lookup(table, idx)


# speedup vs baseline: 3.0254x; 3.0254x over previous
import functools

import jax
import jax.numpy as jnp
from jax import lax
from jax.experimental import pallas as pl
from jax.experimental.pallas import tpu as pltpu
from jax.experimental.pallas import tpu_sc as plsc

NUM_CORES = 2
NUM_SUBCORES = 16
NUM_WORKERS = NUM_CORES * NUM_SUBCORES
LANES = 16
CHUNK = 256


@jax.jit
def kernel(hyperparameters, table):
    num_emb, em_size = table.shape
    batch = hyperparameters.shape[0]
    idx = jnp.squeeze(hyperparameters, axis=1).astype(jnp.int32)
    idx2 = idx >> 1
    parity = idx & 1
    wide_table = table.reshape(num_emb // 2, 2 * em_size)
    b_per_w = batch // NUM_WORKERS

    mesh = plsc.VectorSubcoreMesh(core_axis_name="c", subcore_axis_name="s")

    @functools.partial(
        pl.kernel,
        mesh=mesh,
        out_type=jax.ShapeDtypeStruct((batch, em_size), jnp.float32),
        scratch_types=[
            pltpu.VMEM((b_per_w,), jnp.int32),
            pltpu.VMEM((b_per_w,), jnp.int32),
            pltpu.VMEM((CHUNK, 2 * em_size), jnp.float32),
            pltpu.VMEM((CHUNK, em_size), jnp.float32),
            pltpu.SemaphoreType.DMA,
        ],
    )
    def emb_lookup(table_hbm, idx_hbm, par_hbm, out_hbm, idx_v, par_v, rows_v,
                   out_v, sem):
        wid = lax.axis_index("s") * NUM_CORES + lax.axis_index("c")
        base = wid * b_per_w
        pltpu.sync_copy(idx_hbm.at[pl.ds(base, b_per_w)], idx_v)
        pltpu.sync_copy(par_hbm.at[pl.ds(base, b_per_w)], par_v)

        @pl.loop(0, b_per_w, step=CHUNK)
        def _(c0):
            @pl.loop(0, CHUNK, step=LANES)
            def _(k):
                v = idx_v[pl.ds(c0 + k, LANES)]
                pltpu.async_copy(
                    table_hbm.at[v], rows_v.at[pl.ds(k, LANES)], sem
                )

            pltpu.make_async_copy(
                table_hbm.at[pl.ds(0, CHUNK)], rows_v, sem
            ).wait()

            @pl.loop(0, CHUNK, step=LANES)
            def _(k):
                pv = par_v[pl.ds(c0 + k, LANES)]
                for j in range(LANES):
                    off = pv[j] * em_size
                    for c in range(0, em_size, LANES):
                        out_v[k + j, pl.ds(c, LANES)] = (
                            rows_v[k + j, pl.ds(off + c, LANES)]
                        )

            pltpu.sync_copy(out_v, out_hbm.at[pl.ds(base + c0, CHUNK)])

    return emb_lookup(wide_table, idx2, parity)


# speedup vs baseline: 3.5356x; 1.1687x over previous
import functools

import jax
import jax.numpy as jnp
from jax import lax
from jax.experimental import pallas as pl
from jax.experimental.pallas import tpu as pltpu
from jax.experimental.pallas import tpu_sc as plsc

NUM_CORES = 2
NUM_SUBCORES = 16
NUM_WORKERS = NUM_CORES * NUM_SUBCORES
LANES = 16
CHUNK = 256


@jax.jit
def kernel(hyperparameters, table):
    num_emb, em_size = table.shape
    batch = hyperparameters.shape[0]
    idx = jnp.squeeze(hyperparameters, axis=1).astype(jnp.int32)
    padded = jnp.pad(table, ((0, 0), (0, 128 - em_size)))
    b_per_w = batch // NUM_WORKERS

    mesh = plsc.VectorSubcoreMesh(core_axis_name="c", subcore_axis_name="s")

    @functools.partial(
        pl.kernel,
        mesh=mesh,
        out_type=jax.ShapeDtypeStruct((batch, em_size), jnp.float32),
        scratch_types=[
            pltpu.VMEM((b_per_w,), jnp.int32),
            pltpu.VMEM((CHUNK, 128), jnp.float32),
            pltpu.VMEM((CHUNK, em_size), jnp.float32),
            pltpu.SemaphoreType.DMA,
        ],
    )
    def emb_lookup(table_hbm, idx_hbm, out_hbm, idx_v, rows_v, out_v, sem):
        wid = lax.axis_index("s") * NUM_CORES + lax.axis_index("c")
        base = wid * b_per_w
        pltpu.sync_copy(idx_hbm.at[pl.ds(base, b_per_w)], idx_v)

        @pl.loop(0, b_per_w, step=CHUNK)
        def _(c0):
            @pl.loop(0, CHUNK, step=LANES)
            def _(k):
                v = idx_v[pl.ds(c0 + k, LANES)]
                pltpu.async_copy(
                    table_hbm.at[v], rows_v.at[pl.ds(k, LANES)], sem
                )

            pltpu.make_async_copy(
                table_hbm.at[pl.ds(0, CHUNK)], rows_v, sem
            ).wait()

            @pl.loop(0, CHUNK, step=1)
            def _(r):
                for c in range(0, em_size, LANES):
                    out_v[r, pl.ds(c, LANES)] = rows_v[r, pl.ds(c, LANES)]

            pltpu.sync_copy(out_v, out_hbm.at[pl.ds(base + c0, CHUNK)])

    return emb_lookup(padded, idx)
